# baseline (device time: 4407324 ns/iter reference)
import jax
import jax.numpy as jnp
from jax import lax
from jax.experimental import pallas as pl
from jax.experimental.pallas import tpu as pltpu

N_DEV = 4


def _ring_allgather(x_shard):
    m_per, k = x_shard.shape

    def body(x_ref, out_ref, local_sem, send_sems, recv_sems):
        my_pos = lax.axis_index("i")
        left = (my_pos - 1) % N_DEV
        right = (my_pos + 1) % N_DEV

        barrier_sem = pltpu.get_barrier_semaphore()
        for nbr in [left, right]:
            pl.semaphore_signal(
                barrier_sem, inc=1,
                device_id=(nbr,), device_id_type=pl.DeviceIdType.MESH,
            )
        pl.semaphore_wait(barrier_sem, 2)

        cp = pltpu.make_async_copy(x_ref, out_ref.at[my_pos], local_sem)
        cp.start()
        cp.wait()

        for h in range(N_DEV - 1):
            slot = (my_pos - h) % N_DEV
            rdma = pltpu.make_async_remote_copy(
                src_ref=out_ref.at[slot],
                dst_ref=out_ref.at[slot],
                send_sem=send_sems.at[h],
                recv_sem=recv_sems.at[h],
                device_id=(right,),
                device_id_type=pl.DeviceIdType.MESH,
            )
            rdma.start()
            rdma.wait()

    return pl.pallas_call(
        body,
        out_shape=jax.ShapeDtypeStruct((N_DEV, m_per, k), x_shard.dtype),
        in_specs=[pl.BlockSpec(memory_space=pl.ANY)],
        out_specs=pl.BlockSpec(memory_space=pl.ANY),
        scratch_shapes=[
            pltpu.SemaphoreType.DMA,
            pltpu.SemaphoreType.DMA((N_DEV - 1,)),
            pltpu.SemaphoreType.DMA((N_DEV - 1,)),
        ],
        compiler_params=pltpu.CompilerParams(collective_id=0),
    )(x_shard)


def kernel(x, w_mat):
    m_per, k = x.shape
    x_full = _ring_allgather(x).reshape(N_DEV * m_per, k)
    y = x_full @ w_mat
    c = 0.7978845608028654
    return 0.5 * y * (1.0 + jnp.tanh(c * (y + 0.044715 * y * y * y)))


# device time: 1165161 ns/iter; 3.7826x vs baseline; 3.7826x over previous
import jax
import jax.numpy as jnp
from jax import lax
from jax.experimental import pallas as pl
from jax.experimental.pallas import tpu as pltpu

N_DEV = 4
KT = 1024


def kernel(x, w_mat):
    m_per, k_dim = x.shape
    n_per = w_mat.shape[1]
    nk = k_dim // KT
    half = m_per // 2

    def body(x_ref, w_ref, out_ref, xfull, xbuf, wbuf, acc,
             send_sems, recv_sems, copy_sems, local_sem):
        me = lax.axis_index("i")
        left = (me - 1) % N_DEV
        right = (me + 1) % N_DEV
        opp = (me + 2) % N_DEV

        barrier_sem = pltpu.get_barrier_semaphore()
        for nbr in [left, right]:
            pl.semaphore_signal(
                barrier_sem, inc=1,
                device_id=(nbr,), device_id_type=pl.DeviceIdType.MESH,
            )
        pl.semaphore_wait(barrier_sem, 2)

        cw0 = pltpu.make_async_remote_copy(
            src_ref=x_ref, dst_ref=xfull.at[me],
            send_sem=send_sems.at[0], recv_sem=recv_sems.at[0],
            device_id=(right,), device_id_type=pl.DeviceIdType.MESH)
        ccw0 = pltpu.make_async_remote_copy(
            src_ref=x_ref, dst_ref=xfull.at[me],
            send_sem=send_sems.at[1], recv_sem=recv_sems.at[1],
            device_id=(left,), device_id_type=pl.DeviceIdType.MESH)
        cw0.start()
        ccw0.start()

        def compute_chunk(chunk_ref, slot):
            def mk_copies(ki, buf):
                cx = pltpu.make_async_copy(
                    chunk_ref.at[:, pl.ds(ki * KT, KT)],
                    xbuf.at[buf], copy_sems.at[buf])
                cw = pltpu.make_async_copy(
                    w_ref.at[pl.ds(ki * KT, KT), :],
                    wbuf.at[buf], copy_sems.at[2 + buf])
                return cx, cw

            def start_copies(ki, buf):
                cx, cw = mk_copies(ki, buf)
                cx.start()
                cw.start()

            start_copies(0, 0)

            def step(ki, carry):
                buf = lax.rem(ki, 2)

                @pl.when(ki + 1 < nk)
                def _():
                    start_copies(ki + 1, 1 - buf)

                cx, cw = mk_copies(ki, buf)
                cx.wait()
                cw.wait()
                xa = xbuf[buf].astype(jnp.bfloat16)
                wa = wbuf[buf].astype(jnp.bfloat16)
                p = jnp.dot(xa, wa, preferred_element_type=jnp.float32)

                @pl.when(ki == 0)
                def _():
                    acc[...] = p

                @pl.when(ki > 0)
                def _():
                    acc[...] += p

                return carry

            lax.fori_loop(0, nk, step, 0)
            y = acc[...]
            c = 0.7978845608028654
            acc[...] = 0.5 * y * (1.0 + jnp.tanh(c * (y + 0.044715 * y * y * y)))
            ocp = pltpu.make_async_copy(
                acc, out_ref.at[pl.ds(slot * m_per, m_per), :], local_sem)
            ocp.start()
            ocp.wait()

        compute_chunk(x_ref, me)

        cw0.wait_recv()
        ccw0.wait_recv()

        cw1 = pltpu.make_async_remote_copy(
            src_ref=xfull.at[left, pl.ds(0, half)],
            dst_ref=xfull.at[left, pl.ds(0, half)],
            send_sem=send_sems.at[2], recv_sem=recv_sems.at[2],
            device_id=(right,), device_id_type=pl.DeviceIdType.MESH)
        ccw1 = pltpu.make_async_remote_copy(
            src_ref=xfull.at[right, pl.ds(half, half)],
            dst_ref=xfull.at[right, pl.ds(half, half)],
            send_sem=send_sems.at[3], recv_sem=recv_sems.at[3],
            device_id=(left,), device_id_type=pl.DeviceIdType.MESH)
        cw1.start()
        ccw1.start()

        compute_chunk(xfull.at[left], left)
        compute_chunk(xfull.at[right], right)

        cw1.wait_recv()
        ccw1.wait_recv()
        compute_chunk(xfull.at[opp], opp)

        cw0.wait_send()
        ccw0.wait_send()
        cw1.wait_send()
        ccw1.wait_send()

    out, _xfull = pl.pallas_call(
        body,
        out_shape=[
            jax.ShapeDtypeStruct((N_DEV * m_per, n_per), jnp.float32),
            jax.ShapeDtypeStruct((N_DEV, m_per, k_dim), jnp.float32),
        ],
        in_specs=[
            pl.BlockSpec(memory_space=pl.ANY),
            pl.BlockSpec(memory_space=pl.ANY),
        ],
        out_specs=[
            pl.BlockSpec(memory_space=pl.ANY),
            pl.BlockSpec(memory_space=pl.ANY),
        ],
        scratch_shapes=[
            pltpu.VMEM((2, m_per, KT), jnp.float32),
            pltpu.VMEM((2, KT, n_per), jnp.float32),
            pltpu.VMEM((m_per, n_per), jnp.float32),
            pltpu.SemaphoreType.DMA((4,)),
            pltpu.SemaphoreType.DMA((4,)),
            pltpu.SemaphoreType.DMA((4,)),
            pltpu.SemaphoreType.DMA,
        ],
        compiler_params=pltpu.CompilerParams(
            collective_id=0,
            vmem_limit_bytes=60 * 1024 * 1024,
        ),
    )(x, w_mat)
    return out


# device time: 674552 ns/iter; 6.5337x vs baseline; 1.7273x over previous
import jax
import jax.numpy as jnp
from jax import lax
from jax.experimental import pallas as pl
from jax.experimental.pallas import tpu as pltpu

N_DEV = 4
KT = 1024


def kernel(x, w_mat):
    m_per, k_dim = x.shape
    n_per = w_mat.shape[1]
    nk = k_dim // KT
    half = m_per // 2

    def body(x_ref, w_ref, out_ref, xfull, xbuf, wbuf, acc,
             send_sems, recv_sems, copy_sems, local_sem):
        me = lax.axis_index("i")
        left = (me - 1) % N_DEV
        right = (me + 1) % N_DEV
        opp = (me + 2) % N_DEV

        barrier_sem = pltpu.get_barrier_semaphore()
        for nbr in [left, right]:
            pl.semaphore_signal(
                barrier_sem, inc=1,
                device_id=(nbr,), device_id_type=pl.DeviceIdType.MESH,
            )
        pl.semaphore_wait(barrier_sem, 2)

        cw0 = pltpu.make_async_remote_copy(
            src_ref=x_ref, dst_ref=xfull.at[me],
            send_sem=send_sems.at[0], recv_sem=recv_sems.at[0],
            device_id=(right,), device_id_type=pl.DeviceIdType.MESH)
        ccw0 = pltpu.make_async_remote_copy(
            src_ref=x_ref, dst_ref=xfull.at[me],
            send_sem=send_sems.at[1], recv_sem=recv_sems.at[1],
            device_id=(left,), device_id_type=pl.DeviceIdType.MESH)
        cw0.start()
        ccw0.start()

        def compute_chunk(chunk_ref, slot):
            def mk_copies(ki, buf):
                cx = pltpu.make_async_copy(
                    chunk_ref.at[:, pl.ds(ki * KT, KT)],
                    xbuf.at[buf], copy_sems.at[buf])
                cw = pltpu.make_async_copy(
                    w_ref.at[pl.ds(ki * KT, KT), :],
                    wbuf.at[buf], copy_sems.at[2 + buf])
                return cx, cw

            def start_copies(ki, buf):
                cx, cw = mk_copies(ki, buf)
                cx.start()
                cw.start()

            start_copies(0, 0)

            def step(ki, carry):
                buf = lax.rem(ki, 2)

                @pl.when(ki + 1 < nk)
                def _():
                    start_copies(ki + 1, 1 - buf)

                cx, cw = mk_copies(ki, buf)
                cx.wait()
                cw.wait()
                p = jnp.dot(xbuf[buf], wbuf[buf],
                            preferred_element_type=jnp.float32)

                @pl.when(ki == 0)
                def _():
                    acc[...] = p

                @pl.when(ki > 0)
                def _():
                    acc[...] += p

                return carry

            lax.fori_loop(0, nk, step, 0)
            y = acc[...]
            c = 0.7978845608028654
            acc[...] = 0.5 * y * (1.0 + jnp.tanh(c * (y + 0.044715 * y * y * y)))
            ocp = pltpu.make_async_copy(
                acc, out_ref.at[pl.ds(slot * m_per, m_per), :], local_sem)
            ocp.start()
            ocp.wait()

        compute_chunk(x_ref, me)

        cw0.wait_recv()
        ccw0.wait_recv()

        cw1 = pltpu.make_async_remote_copy(
            src_ref=xfull.at[left, pl.ds(0, half)],
            dst_ref=xfull.at[left, pl.ds(0, half)],
            send_sem=send_sems.at[2], recv_sem=recv_sems.at[2],
            device_id=(right,), device_id_type=pl.DeviceIdType.MESH)
        ccw1 = pltpu.make_async_remote_copy(
            src_ref=xfull.at[right, pl.ds(half, half)],
            dst_ref=xfull.at[right, pl.ds(half, half)],
            send_sem=send_sems.at[3], recv_sem=recv_sems.at[3],
            device_id=(left,), device_id_type=pl.DeviceIdType.MESH)
        cw1.start()
        ccw1.start()

        compute_chunk(xfull.at[left], left)
        compute_chunk(xfull.at[right], right)

        cw1.wait_recv()
        ccw1.wait_recv()
        compute_chunk(xfull.at[opp], opp)

        cw0.wait_send()
        ccw0.wait_send()
        cw1.wait_send()
        ccw1.wait_send()

    out, _xfull = pl.pallas_call(
        body,
        out_shape=[
            jax.ShapeDtypeStruct((N_DEV * m_per, n_per), jnp.float32),
            jax.ShapeDtypeStruct((N_DEV, m_per, k_dim), jnp.bfloat16),
        ],
        in_specs=[
            pl.BlockSpec(memory_space=pl.ANY),
            pl.BlockSpec(memory_space=pl.ANY),
        ],
        out_specs=[
            pl.BlockSpec(memory_space=pl.ANY),
            pl.BlockSpec(memory_space=pl.ANY),
        ],
        scratch_shapes=[
            pltpu.VMEM((2, m_per, KT), jnp.bfloat16),
            pltpu.VMEM((2, KT, n_per), jnp.bfloat16),
            pltpu.VMEM((m_per, n_per), jnp.float32),
            pltpu.SemaphoreType.DMA((4,)),
            pltpu.SemaphoreType.DMA((4,)),
            pltpu.SemaphoreType.DMA((4,)),
            pltpu.SemaphoreType.DMA,
        ],
        compiler_params=pltpu.CompilerParams(
            collective_id=0,
            vmem_limit_bytes=60 * 1024 * 1024,
        ),
    )(x.astype(jnp.bfloat16), w_mat.astype(jnp.bfloat16))
    return out


# device time: 460755 ns/iter; 9.5654x vs baseline; 1.4640x over previous
import jax
import jax.numpy as jnp
from jax import lax
from jax.experimental import pallas as pl
from jax.experimental.pallas import tpu as pltpu

N_DEV = 4
KT = 1024


def kernel(x, w_mat):
    m_per, k_dim = x.shape
    n_per = w_mat.shape[1]
    nk = k_dim // KT
    khalf = k_dim // 2

    def body(x_ref, w_ref, out_ref, wfull,
             xbuf, wbuf, acc, bsend, brecv,
             wsend_sems, wrecv_sems, bsend_sems, brecv_sems,
             copy_sems, local_sem):
        me = lax.axis_index("i")
        left = (me - 1) % N_DEV
        right = (me + 1) % N_DEV
        opp = (me + 2) % N_DEV

        barrier_sem = pltpu.get_barrier_semaphore()
        for nbr in [left, right, opp]:
            pl.semaphore_signal(
                barrier_sem, inc=1,
                device_id=(nbr,), device_id_type=pl.DeviceIdType.MESH,
            )
        pl.semaphore_wait(barrier_sem, 3)

        wr0 = pltpu.make_async_remote_copy(
            src_ref=w_ref, dst_ref=wfull.at[me],
            send_sem=wsend_sems.at[0], recv_sem=wrecv_sems.at[0],
            device_id=(right,), device_id_type=pl.DeviceIdType.MESH)
        wl0 = pltpu.make_async_remote_copy(
            src_ref=w_ref, dst_ref=wfull.at[me],
            send_sem=wsend_sems.at[1], recv_sem=wrecv_sems.at[1],
            device_id=(left,), device_id_type=pl.DeviceIdType.MESH)
        wr0.start()
        wl0.start()

        def compute_block(wsrc_ref):
            def mk_copies(ki, buf):
                cx = pltpu.make_async_copy(
                    x_ref.at[:, pl.ds(ki * KT, KT)],
                    xbuf.at[buf], copy_sems.at[buf])
                cw = pltpu.make_async_copy(
                    wsrc_ref.at[pl.ds(ki * KT, KT), :],
                    wbuf.at[buf], copy_sems.at[2 + buf])
                return cx, cw

            def start_copies(ki, buf):
                cx, cw = mk_copies(ki, buf)
                cx.start()
                cw.start()

            start_copies(0, 0)

            def step(ki, carry):
                buf = lax.rem(ki, 2)

                @pl.when(ki + 1 < nk)
                def _():
                    start_copies(ki + 1, 1 - buf)

                cx, cw = mk_copies(ki, buf)
                cx.wait()
                cw.wait()
                p = jnp.dot(xbuf[buf], wbuf[buf],
                            preferred_element_type=jnp.float32)

                @pl.when(ki == 0)
                def _():
                    acc[...] = p

                @pl.when(ki > 0)
                def _():
                    acc[...] += p

                return carry

            lax.fori_loop(0, nk, step, 0)
            y = acc[...]
            c = 0.7978845608028654
            acc[...] = 0.5 * y * (1.0 + jnp.tanh(c * (y + 0.044715 * y * y * y)))

        compute_block(w_ref)
        own_cp = pltpu.make_async_copy(
            acc, out_ref.at[pl.ds(me * m_per, m_per), :], local_sem)
        own_cp.start()
        own_cp.wait()

        wr0.wait_recv()
        wl0.wait_recv()

        wr1 = pltpu.make_async_remote_copy(
            src_ref=wfull.at[left, pl.ds(0, khalf)],
            dst_ref=wfull.at[left, pl.ds(0, khalf)],
            send_sem=wsend_sems.at[2], recv_sem=wrecv_sems.at[2],
            device_id=(right,), device_id_type=pl.DeviceIdType.MESH)
        wl1 = pltpu.make_async_remote_copy(
            src_ref=wfull.at[right, pl.ds(khalf, khalf)],
            dst_ref=wfull.at[right, pl.ds(khalf, khalf)],
            send_sem=wsend_sems.at[3], recv_sem=wrecv_sems.at[3],
            device_id=(left,), device_id_type=pl.DeviceIdType.MESH)
        wr1.start()
        wl1.start()

        compute_block(wfull.at[right])
        bsend[0] = acc[...].astype(jnp.bfloat16)
        b_r = pltpu.make_async_remote_copy(
            src_ref=bsend.at[0], dst_ref=brecv.at[0],
            send_sem=bsend_sems.at[0], recv_sem=brecv_sems.at[0],
            device_id=(right,), device_id_type=pl.DeviceIdType.MESH)
        b_r.start()

        compute_block(wfull.at[left])
        bsend[1] = acc[...].astype(jnp.bfloat16)
        b_l = pltpu.make_async_remote_copy(
            src_ref=bsend.at[1], dst_ref=brecv.at[1],
            send_sem=bsend_sems.at[1], recv_sem=brecv_sems.at[1],
            device_id=(left,), device_id_type=pl.DeviceIdType.MESH)
        b_l.start()

        wr1.wait_recv()
        wl1.wait_recv()

        compute_block(wfull.at[opp])
        bsend[2] = acc[...].astype(jnp.bfloat16)
        b_o = pltpu.make_async_remote_copy(
            src_ref=bsend.at[2], dst_ref=brecv.at[2],
            send_sem=bsend_sems.at[2], recv_sem=brecv_sems.at[2],
            device_id=(opp,), device_id_type=pl.DeviceIdType.MESH)
        b_o.start()

        for s, origin in [(0, left), (1, right), (2, opp)]:
            rwait = pltpu.make_async_remote_copy(
                src_ref=bsend.at[s], dst_ref=brecv.at[s],
                send_sem=bsend_sems.at[s], recv_sem=brecv_sems.at[s],
                device_id=(me,), device_id_type=pl.DeviceIdType.MESH)
            rwait.wait_recv()
            acc[...] = brecv[s].astype(jnp.float32)
            st = pltpu.make_async_copy(
                acc, out_ref.at[pl.ds(origin * m_per, m_per), :], local_sem)
            st.start()
            st.wait()

        wr0.wait_send()
        wl0.wait_send()
        wr1.wait_send()
        wl1.wait_send()
        b_r.wait_send()
        b_l.wait_send()
        b_o.wait_send()

    out, _wfull = pl.pallas_call(
        body,
        out_shape=[
            jax.ShapeDtypeStruct((N_DEV * m_per, n_per), jnp.float32),
            jax.ShapeDtypeStruct((N_DEV, k_dim, n_per), jnp.bfloat16),
        ],
        in_specs=[
            pl.BlockSpec(memory_space=pl.ANY),
            pl.BlockSpec(memory_space=pl.ANY),
        ],
        out_specs=[
            pl.BlockSpec(memory_space=pl.ANY),
            pl.BlockSpec(memory_space=pl.ANY),
        ],
        scratch_shapes=[
            pltpu.VMEM((2, m_per, KT), jnp.bfloat16),
            pltpu.VMEM((2, KT, n_per), jnp.bfloat16),
            pltpu.VMEM((m_per, n_per), jnp.float32),
            pltpu.VMEM((3, m_per, n_per), jnp.bfloat16),
            pltpu.VMEM((3, m_per, n_per), jnp.bfloat16),
            pltpu.SemaphoreType.DMA((4,)),
            pltpu.SemaphoreType.DMA((4,)),
            pltpu.SemaphoreType.DMA((3,)),
            pltpu.SemaphoreType.DMA((3,)),
            pltpu.SemaphoreType.DMA((4,)),
            pltpu.SemaphoreType.DMA,
        ],
        compiler_params=pltpu.CompilerParams(
            collective_id=0,
            vmem_limit_bytes=60 * 1024 * 1024,
        ),
    )(x.astype(jnp.bfloat16), w_mat.astype(jnp.bfloat16))
    return out
